# baseline (speedup 1.0000x reference)
import functools

import jax
import jax.numpy as jnp
from jax import lax
from jax.experimental import pallas as pl
from jax.experimental.pallas import tpu as pltpu
from jax.experimental.pallas import tpu_sc as plsc

B, N1, N2 = 16, 2048, 512
C1, C2 = 64, 128
K = 8
QT = 2048


def _precompute_h_kernel(feat2_ref, xyz2_ref, wf_ref, wx_ref, b_ref, h_ref):
    f2 = feat2_ref[0]
    x2 = xyz2_ref[0]
    h = jnp.dot(f2, wf_ref[...], preferred_element_type=jnp.float32)
    h = h + jnp.dot(x2, wx_ref[...], preferred_element_type=jnp.float32)
    h_ref[0] = h + b_ref[...]


def _knn_kernel(xyz1_ref, xyz2t_ref, idx_ref):
    b = pl.program_id(0)
    x1 = xyz1_ref[0]
    x2t = xyz2t_ref[0]
    d = None
    for c in range(3):
        dc = x1[:, c:c + 1] - x2t[c:c + 1, :]
        dc = dc * dc
        d = dc if d is None else d + dc
    lane = jax.lax.broadcasted_iota(jnp.int32, (QT, N2), 1)
    for t in range(K):
        jidx = jnp.argmin(d, axis=1)[:, None]
        sel = lane == jidx
        d = jnp.where(sel, jnp.float32(3e38), d)
        idx_ref[0, :, t] = jidx[:, 0] + b * N2


def _mlp_kernel(g_ref, xyz1_ref, feat1_ref,
                wx_ref, w11_ref, b11_ref, w12_ref, b12_ref,
                w2a_ref, w2b_ref, b2_ref, out_ref):
    x1 = xyz1_ref[0]
    qoff = jnp.dot(x1, wx_ref[...], preferred_element_type=jnp.float32)
    pooled = None
    for t in range(K):
        l1 = g_ref[0, :, t, :64]
        l1 = jnp.maximum(l1 - qoff, 0.0)
        l2 = jnp.dot(l1, w11_ref[...], preferred_element_type=jnp.float32) + b11_ref[...]
        l2 = jnp.maximum(l2, 0.0)
        l3 = jnp.dot(l2, w12_ref[...], preferred_element_type=jnp.float32) + b12_ref[...]
        l3 = jnp.maximum(l3, 0.0)
        pooled = l3 if pooled is None else jnp.maximum(pooled, l3)
    out = jnp.dot(pooled, w2a_ref[...], preferred_element_type=jnp.float32)
    out = out + jnp.dot(feat1_ref[0], w2b_ref[...], preferred_element_type=jnp.float32)
    out = out + b2_ref[...]
    out_ref[0] = jnp.maximum(out, 0.0)


def _sc_gather(table, idx):
    R = idx.shape[0]
    info = plsc.get_sparse_core_info()
    NW = info.num_cores * info.num_subcores
    r_per_w = R // NW
    CH = 512
    mesh = plsc.VectorSubcoreMesh(core_axis_name="c", subcore_axis_name="s")

    @functools.partial(
        pl.kernel, mesh=mesh,
        out_type=jax.ShapeDtypeStruct((R, 128), jnp.float32),
        scratch_types=[
            pltpu.VMEM((CH,), jnp.int32),
            pltpu.VMEM((CH, 128), jnp.float32),
            pltpu.SemaphoreType.DMA,
        ],
    )
    def k(table_hbm, idx_hbm, out_hbm, idx_v, rows_v, sem):
        wid = lax.axis_index("s") * info.num_cores + lax.axis_index("c")
        base = wid * r_per_w
        for chunk in range(r_per_w // CH):
            off = base + chunk * CH
            pltpu.sync_copy(idx_hbm.at[pl.ds(off, CH)], idx_v)
            pltpu.async_copy(table_hbm.at[idx_v], rows_v, sem).wait()
            pltpu.sync_copy(rows_v, out_hbm.at[pl.ds(off, CH)])

    return k(table, idx)


@jax.jit
def kernel(xyz1, feat1, xyz2, feat2, W1_0, b1_0, W1_1, b1_1, W1_2, b1_2, W2_0, b2_0):
    wf = W1_0[:C2]
    wx = W1_0[C2:]
    w2a = W2_0[:128]
    w2b = W2_0[128:]
    b1_0r = b1_0.reshape(1, -1)
    b11 = b1_1.reshape(1, -1)
    b12 = b1_2.reshape(1, -1)
    b2 = b2_0.reshape(1, -1)
    xyz2t = jnp.transpose(xyz2, (0, 2, 1))

    h = pl.pallas_call(
        _precompute_h_kernel,
        grid=(B,),
        in_specs=[
            pl.BlockSpec((1, N2, C2), lambda b: (b, 0, 0)),
            pl.BlockSpec((1, N2, 3), lambda b: (b, 0, 0)),
            pl.BlockSpec((C2, 64), lambda b: (0, 0)),
            pl.BlockSpec((3, 64), lambda b: (0, 0)),
            pl.BlockSpec((1, 64), lambda b: (0, 0)),
        ],
        out_specs=pl.BlockSpec((1, N2, 64), lambda b: (b, 0, 0)),
        out_shape=jax.ShapeDtypeStruct((B, N2, 64), jnp.float32),
        compiler_params=pltpu.CompilerParams(
            dimension_semantics=("parallel",),
        ),
    )(feat2, xyz2, wf, wx, b1_0r)

    idx = pl.pallas_call(
        _knn_kernel,
        grid=(B,),
        in_specs=[
            pl.BlockSpec((1, QT, 3), lambda b: (b, 0, 0)),
            pl.BlockSpec((1, 3, N2), lambda b: (b, 0, 0)),
        ],
        out_specs=pl.BlockSpec((1, QT, K), lambda b: (b, 0, 0)),
        out_shape=jax.ShapeDtypeStruct((B, N1, K), jnp.int32),
        compiler_params=pltpu.CompilerParams(
            dimension_semantics=("arbitrary",),
        ),
    )(xyz1, xyz2t)

    h_pad = jnp.pad(h, ((0, 0), (0, 0), (0, 64)))
    g = _sc_gather(h_pad.reshape(B * N2, 128), idx.reshape(B * N1 * K))
    g = g.reshape(B, N1, K, 128)

    out = pl.pallas_call(
        _mlp_kernel,
        grid=(B,),
        in_specs=[
            pl.BlockSpec((1, QT, K, 128), lambda b: (b, 0, 0, 0)),
            pl.BlockSpec((1, QT, 3), lambda b: (b, 0, 0)),
            pl.BlockSpec((1, QT, C1), lambda b: (b, 0, 0)),
            pl.BlockSpec((3, 64), lambda b: (0, 0)),
            pl.BlockSpec((64, 64), lambda b: (0, 0)),
            pl.BlockSpec((1, 64), lambda b: (0, 0)),
            pl.BlockSpec((64, 128), lambda b: (0, 0)),
            pl.BlockSpec((1, 128), lambda b: (0, 0)),
            pl.BlockSpec((128, 128), lambda b: (0, 0)),
            pl.BlockSpec((64, 128), lambda b: (0, 0)),
            pl.BlockSpec((1, 128), lambda b: (0, 0)),
        ],
        out_specs=pl.BlockSpec((1, QT, 128), lambda b: (b, 0, 0)),
        out_shape=jax.ShapeDtypeStruct((B, N1, 128), jnp.float32),
        compiler_params=pltpu.CompilerParams(
            dimension_semantics=("arbitrary",),
        ),
    )(g, xyz1, feat1, wx, W1_1, b11, W1_2, b12, w2a, w2b, b2)

    return out


# speedup vs baseline: 1.9074x; 1.9074x over previous
import jax
import jax.numpy as jnp
from jax.experimental import pallas as pl
from jax.experimental.pallas import tpu as pltpu

B, N1, N2 = 16, 2048, 512
C1, C2 = 64, 128
K = 8
QT = 2048


def _main_kernel(xyz1_ref, xyz2t_ref, feat2_ref, xyz2_ref, feat1_ref,
                 wf_ref, wx_ref, b10_ref, w11_ref, b11_ref, w12_ref, b12_ref,
                 w2a_ref, w2b_ref, b2_ref, out_ref):
    x1 = xyz1_ref[0]
    x2t = xyz2t_ref[0]

    h = jnp.dot(feat2_ref[0], wf_ref[...], preferred_element_type=jnp.float32)
    h = h + jnp.dot(xyz2_ref[0], wx_ref[...], preferred_element_type=jnp.float32)
    h = h + b10_ref[...]

    d = None
    for c in range(3):
        dc = x1[:, c:c + 1] - x2t[c:c + 1, :]
        dc = dc * dc
        d = dc if d is None else d + dc

    lane = jax.lax.broadcasted_iota(jnp.int32, (QT, N2), 1)
    qoff = jnp.dot(x1, wx_ref[...], preferred_element_type=jnp.float32)

    pooled = None
    for _ in range(K):
        jidx = jnp.argmin(d, axis=1)[:, None]
        sel = lane == jidx
        d = jnp.where(sel, jnp.float32(3e38), d)
        oh = sel.astype(jnp.float32)
        l1 = jnp.dot(oh, h, preferred_element_type=jnp.float32)
        l1 = jnp.maximum(l1 - qoff, 0.0)
        l2 = jnp.dot(l1, w11_ref[...], preferred_element_type=jnp.float32) + b11_ref[...]
        l2 = jnp.maximum(l2, 0.0)
        l3 = jnp.dot(l2, w12_ref[...], preferred_element_type=jnp.float32) + b12_ref[...]
        l3 = jnp.maximum(l3, 0.0)
        pooled = l3 if pooled is None else jnp.maximum(pooled, l3)

    out = jnp.dot(pooled, w2a_ref[...], preferred_element_type=jnp.float32)
    out = out + jnp.dot(feat1_ref[0], w2b_ref[...], preferred_element_type=jnp.float32)
    out = out + b2_ref[...]
    out_ref[0] = jnp.maximum(out, 0.0)


@jax.jit
def kernel(xyz1, feat1, xyz2, feat2, W1_0, b1_0, W1_1, b1_1, W1_2, b1_2, W2_0, b2_0):
    wf = W1_0[:C2]
    wx = W1_0[C2:]
    w2a = W2_0[:128]
    w2b = W2_0[128:]
    b1_0r = b1_0.reshape(1, -1)
    b11 = b1_1.reshape(1, -1)
    b12 = b1_2.reshape(1, -1)
    b2 = b2_0.reshape(1, -1)
    xyz2t = jnp.transpose(xyz2, (0, 2, 1))

    out = pl.pallas_call(
        _main_kernel,
        grid=(B, N1 // QT),
        in_specs=[
            pl.BlockSpec((1, QT, 3), lambda b, q: (b, q, 0)),
            pl.BlockSpec((1, 3, N2), lambda b, q: (b, 0, 0)),
            pl.BlockSpec((1, N2, C2), lambda b, q: (b, 0, 0)),
            pl.BlockSpec((1, N2, 3), lambda b, q: (b, 0, 0)),
            pl.BlockSpec((1, QT, C1), lambda b, q: (b, q, 0)),
            pl.BlockSpec((C2, 64), lambda b, q: (0, 0)),
            pl.BlockSpec((3, 64), lambda b, q: (0, 0)),
            pl.BlockSpec((1, 64), lambda b, q: (0, 0)),
            pl.BlockSpec((64, 64), lambda b, q: (0, 0)),
            pl.BlockSpec((1, 64), lambda b, q: (0, 0)),
            pl.BlockSpec((64, 128), lambda b, q: (0, 0)),
            pl.BlockSpec((1, 128), lambda b, q: (0, 0)),
            pl.BlockSpec((128, 128), lambda b, q: (0, 0)),
            pl.BlockSpec((64, 128), lambda b, q: (0, 0)),
            pl.BlockSpec((1, 128), lambda b, q: (0, 0)),
        ],
        out_specs=pl.BlockSpec((1, QT, 128), lambda b, q: (b, q, 0)),
        out_shape=jax.ShapeDtypeStruct((B, N1, 128), jnp.float32),
        compiler_params=pltpu.CompilerParams(
            dimension_semantics=("parallel", "parallel"),
        ),
    )(xyz1, xyz2t, feat2, xyz2, feat1, wf, wx, b1_0r, W1_1, b11, W1_2, b12, w2a, w2b, b2)

    return out
